# baseline (speedup 1.0000x reference)
import functools

import jax
import jax.numpy as jnp
from jax import lax
from jax.experimental import pallas as pl
from jax.experimental.pallas import tpu as pltpu
from jax.experimental.pallas import tpu_sc as plsc

BATCH = 16384
NROWS = 1000000
EMB = 64
LANES = 16
W = 8
VPB = 1
BLANES = VPB * LANES * W

_info = plsc.get_sparse_core_info()
NC = _info.num_cores
NS = _info.num_subcores
NW = NC * NS
BPW = BATCH // NW
NVEC = BPW // LANES
NRND = NVEC // VPB


def _body(u_hbm, v_hbm, ut_hbm, vt_hbm, out_hbm,
          uidx, vidx, w_a, w_b, outv, sem_a, sem_b):
    wid = lax.axis_index("s") * NC + lax.axis_index("c")
    crow = wid * (BPW // 128)

    pltpu.sync_copy(u_hbm.at[pl.ds(crow, BPW // 128)], uidx)
    pltpu.sync_copy(v_hbm.at[pl.ds(crow, BPW // 128)], vidx)

    iota = lax.iota(jnp.int32, LANES)

    def idx_vec(ref, t):
        return ref[t // 8, pl.ds((t % 8) * LANES, LANES)]

    def stage(r, w, sem):
        for t in range(VPB):
            uvec = idx_vec(uidx, r * VPB + t)
            vvec = idx_vec(vidx, r * VPB + t)
            for vec, tab, half in ((uvec, ut_hbm, 0), (vvec, vt_hbm, BLANES)):
                for k in range(LANES):
                    i_k = vec[k]
                    o128 = pl.multiple_of(i_k & -128, 128)
                    o8 = (i_k >> 3 & 15) * W
                    src = tab.at[:, :, pl.ds(o128, 128)].at[:, :, pl.ds(o8, W)]
                    dst = w.at[:, :, pl.ds(half + (t * LANES + k) * W, W)]
                    pltpu.async_copy(src, dst, sem)

    def drain(w, sem):
        dummy = ut_hbm.at[:, :, pl.ds(0, 2 * BLANES)]
        pltpu.make_async_copy(dummy, w, sem).wait()

    def compute(r, w):
        for t in range(VPB):
            base = iota * W + t * (LANES * W)
            offs_u = base + (idx_vec(uidx, r * VPB + t) & (W - 1))
            offs_v = BLANES + base + (idx_vec(vidx, r * VPB + t) & (W - 1))
            acc = [jnp.zeros((LANES,), jnp.float32) for _ in range(4)]
            for j in range(EMB):
                tj = jnp.full((LANES,), j // 8, jnp.int32)
                rw = jnp.full((LANES,), j % 8, jnp.int32)
                uc = plsc.load_gather(w, [tj, rw, offs_u])
                vc = plsc.load_gather(w, [tj, rw, offs_v])
                acc[j % 4] = acc[j % 4] + uc * vc
            outv[pl.ds((r * VPB + t) * LANES, LANES)] = (
                (acc[0] + acc[1]) + (acc[2] + acc[3]))

    stage(0, w_a, sem_a)

    def pipelined(i, _):
        stage(2 * i + 1, w_b, sem_b)
        drain(w_a, sem_a)
        compute(2 * i, w_a)

        stage(2 * i + 2, w_a, sem_a)
        drain(w_b, sem_b)
        compute(2 * i + 1, w_b)
        return 0

    lax.fori_loop(0, NRND // 2 - 1, pipelined, 0)

    stage(NRND - 1, w_b, sem_b)
    drain(w_a, sem_a)
    compute(NRND - 2, w_a)
    drain(w_b, sem_b)
    compute(NRND - 1, w_b)

    pltpu.sync_copy(outv, out_hbm.at[pl.ds(wid * BPW, BPW)])


@functools.partial(
    pl.kernel,
    out_type=jax.ShapeDtypeStruct((BATCH,), jnp.float32),
    mesh=plsc.VectorSubcoreMesh(core_axis_name="c", subcore_axis_name="s"),
    compiler_params=pltpu.CompilerParams(needs_layout_passes=False),
    scratch_types=[
        pltpu.VMEM((BPW // 128, 128), jnp.int32),
        pltpu.VMEM((BPW // 128, 128), jnp.int32),
        pltpu.VMEM((8, 8, 2 * BLANES), jnp.float32),
        pltpu.VMEM((8, 8, 2 * BLANES), jnp.float32),
        pltpu.VMEM((BPW,), jnp.float32),
        pltpu.SemaphoreType.DMA,
        pltpu.SemaphoreType.DMA,
    ],
)
def _mf_kernel(u_hbm, v_hbm, ut_hbm, vt_hbm, out_hbm,
               uidx, vidx, w_a, w_b, outv, sem_a, sem_b):
    _body(u_hbm, v_hbm, ut_hbm, vt_hbm, out_hbm,
          uidx, vidx, w_a, w_b, outv, sem_a, sem_b)


def kernel(u, v, user_emb, item_emb):
    u2 = u.astype(jnp.int32).reshape(BATCH // 128, 128)
    v2 = v.astype(jnp.int32).reshape(BATCH // 128, 128)
    ut = user_emb.T.reshape(8, 8, NROWS)
    vt = item_emb.T.reshape(8, 8, NROWS)
    return _mf_kernel(u2, v2, ut, vt)


# speedup vs baseline: 1.0088x; 1.0088x over previous
import functools

import jax
import jax.numpy as jnp
from jax import lax
from jax.experimental import pallas as pl
from jax.experimental.pallas import tpu as pltpu
from jax.experimental.pallas import tpu_sc as plsc

BATCH = 16384
NROWS = 1000000
EMB = 64
LANES = 16
W = 8
VPB = 1
BLANES = VPB * LANES * W

_info = plsc.get_sparse_core_info()
NC = _info.num_cores
NS = _info.num_subcores
NW = NC * NS
BPW = BATCH // NW
NVEC = BPW // LANES
NRND = NVEC // VPB


def _body(u_hbm, v_hbm, ut_hbm, vt_hbm, out_hbm,
          uidx, vidx, w_a, w_b, outv, sem_a, sem_b):
    wid = lax.axis_index("s") * NC + lax.axis_index("c")
    crow = wid * (BPW // 128)

    pltpu.sync_copy(u_hbm.at[pl.ds(crow, BPW // 128)], uidx)
    pltpu.sync_copy(v_hbm.at[pl.ds(crow, BPW // 128)], vidx)

    iota = lax.iota(jnp.int32, LANES)

    def idx_vec(ref, t):
        return ref[t // 8, pl.ds((t % 8) * LANES, LANES)]

    def stage(r, w, sem):
        for t in range(VPB):
            uvec = idx_vec(uidx, r * VPB + t)
            vvec = idx_vec(vidx, r * VPB + t)
            for k in range(LANES):
                for vec, tab, half in ((uvec, ut_hbm, 0),
                                       (vvec, vt_hbm, BLANES)):
                    i_k = vec[k]
                    o128 = pl.multiple_of(i_k & -128, 128)
                    o8 = (i_k >> 3 & 15) * W
                    src = tab.at[:, :, pl.ds(o128, 128)].at[:, :, pl.ds(o8, W)]
                    dst = w.at[:, :, pl.ds(half + (t * LANES + k) * W, W)]
                    pltpu.async_copy(src, dst, sem)

    def drain(w, sem):
        dummy = ut_hbm.at[:, :, pl.ds(0, 2 * BLANES)]
        pltpu.make_async_copy(dummy, w, sem).wait()

    def compute(r, w):
        for t in range(VPB):
            base = iota * W + t * (LANES * W)
            offs_u = base + (idx_vec(uidx, r * VPB + t) & (W - 1))
            offs_v = BLANES + base + (idx_vec(vidx, r * VPB + t) & (W - 1))
            acc = [jnp.zeros((LANES,), jnp.float32) for _ in range(4)]
            for j in range(EMB):
                tj = jnp.full((LANES,), j // 8, jnp.int32)
                rw = jnp.full((LANES,), j % 8, jnp.int32)
                uc = plsc.load_gather(w, [tj, rw, offs_u])
                vc = plsc.load_gather(w, [tj, rw, offs_v])
                acc[j % 4] = acc[j % 4] + uc * vc
            outv[pl.ds((r * VPB + t) * LANES, LANES)] = (
                (acc[0] + acc[1]) + (acc[2] + acc[3]))

    stage(0, w_a, sem_a)

    def pipelined(i, _):
        stage(2 * i + 1, w_b, sem_b)
        drain(w_a, sem_a)
        compute(2 * i, w_a)

        stage(2 * i + 2, w_a, sem_a)
        drain(w_b, sem_b)
        compute(2 * i + 1, w_b)
        return 0

    lax.fori_loop(0, NRND // 2 - 1, pipelined, 0)

    stage(NRND - 1, w_b, sem_b)
    drain(w_a, sem_a)
    compute(NRND - 2, w_a)
    drain(w_b, sem_b)
    compute(NRND - 1, w_b)

    pltpu.sync_copy(outv, out_hbm.at[pl.ds(wid * BPW, BPW)])


@functools.partial(
    pl.kernel,
    out_type=jax.ShapeDtypeStruct((BATCH,), jnp.float32),
    mesh=plsc.VectorSubcoreMesh(core_axis_name="c", subcore_axis_name="s"),
    compiler_params=pltpu.CompilerParams(needs_layout_passes=False),
    scratch_types=[
        pltpu.VMEM((BPW // 128, 128), jnp.int32),
        pltpu.VMEM((BPW // 128, 128), jnp.int32),
        pltpu.VMEM((8, 8, 2 * BLANES), jnp.float32),
        pltpu.VMEM((8, 8, 2 * BLANES), jnp.float32),
        pltpu.VMEM((BPW,), jnp.float32),
        pltpu.SemaphoreType.DMA,
        pltpu.SemaphoreType.DMA,
    ],
)
def _mf_kernel(u_hbm, v_hbm, ut_hbm, vt_hbm, out_hbm,
               uidx, vidx, w_a, w_b, outv, sem_a, sem_b):
    _body(u_hbm, v_hbm, ut_hbm, vt_hbm, out_hbm,
          uidx, vidx, w_a, w_b, outv, sem_a, sem_b)


def kernel(u, v, user_emb, item_emb):
    u2 = u.astype(jnp.int32).reshape(BATCH // 128, 128)
    v2 = v.astype(jnp.int32).reshape(BATCH // 128, 128)
    ut = user_emb.T.reshape(8, 8, NROWS)
    vt = item_emb.T.reshape(8, 8, NROWS)
    return _mf_kernel(u2, v2, ut, vt)
